# baseline (speedup 1.0000x reference)
import jax
import jax.numpy as jnp
from jax.experimental import pallas as pl
from jax.experimental.pallas import tpu as pltpu

_B = 64
_N = 3000
_NPAD = 3072
_JT = 384
_NJT = _NPAD // _JT
_SYM = (12, 15, 18, 19, 20)
_PAD_COORD = 1.0e6


def _loss_kernel(mask_ref, rtv_ref, mp3_ref, tgt3_ref, out_ref,
                 tfx_s, tfy_s, tfz_s, r2_s):
    col_id = jax.lax.broadcasted_iota(jnp.int32, (1, _NPAD), 1)
    col_ok = col_id < _N
    lane = jnp.ones((1, 128), jnp.float32)

    mpx = mp3_ref[0]
    mpy = mp3_ref[1]
    mpz = mp3_ref[2]

    def c(k):
        return rtv_ref[:, k:k + 1]

    tfx = mpx * c(0) + mpy * c(3) + mpz * c(6) + c(9)
    tfy = mpx * c(1) + mpy * c(4) + mpz * c(7) + c(10)
    tfz = mpx * c(2) + mpy * c(5) + mpz * c(8) + c(11)
    tfx_s[...] = tfx
    tfy_s[...] = tfy
    tfz_s[...] = tfz

    tgx = tgt3_ref[0]
    tgy = tgt3_ref[1]
    tgz = tgt3_ref[2]
    r2_s[...] = tgx * tgx + tgy * tgy + tgz * tgz

    dx = tfx - tgx
    dy = tfy - tgy
    dz = tfz - tgz
    dis = jnp.sqrt(dx * dx + dy * dy + dz * dz)
    dsum = jnp.sum(jnp.where(col_ok, dis, 0.0), axis=1, keepdims=True) / _N
    out_ref[...] = dsum * lane

    def body(b, carry):
        @pl.when(mask_ref[b] != 0)
        def _sym_path():
            bs = pl.ds(b, 1)
            tfxr = tfx_s[bs, :]
            tfyr = tfy_s[bs, :]
            tfzr = tfz_s[bs, :]
            ones = jnp.ones_like(tfxr)
            q2 = tfxr * tfxr + tfyr * tfyr + tfzr * tfzr
            yk = jnp.concatenate(
                [-2.0 * tfxr, -2.0 * tfyr, -2.0 * tfzr, q2, ones], axis=0)
            tgxr = tgt3_ref[0, bs, :]
            tgyr = tgt3_ref[1, bs, :]
            tgzr = tgt3_ref[2, bs, :]
            r2r = r2_s[bs, :]
            dmin = None
            for j in range(_NJT):
                js = slice(j * _JT, (j + 1) * _JT)
                xkT = jnp.concatenate(
                    [tgxr[:, js], tgyr[:, js], tgzr[:, js],
                     ones[:, js], r2r[:, js]], axis=0)
                d2t = jax.lax.dot_general(
                    xkT, yk, (((0,), (0,)), ((), ())),
                    preferred_element_type=jnp.float32)
                tmin = jnp.min(d2t, axis=0, keepdims=True)
                dmin = tmin if dmin is None else jnp.minimum(dmin, tmin)
            disr = jnp.sqrt(jnp.maximum(dmin, 0.0))
            val = jnp.sum(jnp.where(col_ok, disr, 0.0)) / _N
            out_ref[bs, :] = val * lane

        return carry

    jax.lax.fori_loop(0, _B, body, 0)


@jax.jit
def _run(mask, rtv, mp3, tgt3):
    grid_spec = pltpu.PrefetchScalarGridSpec(
        num_scalar_prefetch=1,
        grid=(1,),
        in_specs=[
            pl.BlockSpec((_B, 16), lambda g, m: (0, 0)),
            pl.BlockSpec((3, _B, _NPAD), lambda g, m: (0, 0, 0)),
            pl.BlockSpec((3, _B, _NPAD), lambda g, m: (0, 0, 0)),
        ],
        out_specs=pl.BlockSpec((_B, 128), lambda g, m: (0, 0)),
        scratch_shapes=[pltpu.VMEM((_B, _NPAD), jnp.float32)] * 4,
    )
    return pl.pallas_call(
        _loss_kernel,
        grid_spec=grid_spec,
        out_shape=jax.ShapeDtypeStruct((_B, 128), jnp.float32),
        compiler_params=pltpu.CompilerParams(
            dimension_semantics=("arbitrary",),
        ),
    )(mask, rtv, mp3, tgt3)


def kernel(pred_r, pred_t, target, model_points, idx):
    pred_r = pred_r / jnp.linalg.norm(pred_r, axis=1, keepdims=True)
    w, x, y, z = pred_r[:, 0], pred_r[:, 1], pred_r[:, 2], pred_r[:, 3]
    rt = jnp.stack([
        1.0 - 2.0 * (y * y + z * z), 2.0 * (x * y - w * z), 2.0 * (x * z + w * y),
        2.0 * (x * y + w * z), 1.0 - 2.0 * (x * x + z * z), 2.0 * (y * z - w * x),
        2.0 * (x * z - w * y), 2.0 * (y * z + w * x), 1.0 - 2.0 * (x * x + y * y),
        pred_t[:, 0], pred_t[:, 1], pred_t[:, 2],
    ], axis=1)
    rtv = jnp.pad(rt, ((0, 0), (0, 4)))

    sym = jnp.asarray(_SYM, dtype=idx.dtype)
    mask = (idx[:, 0][:, None] == sym[None, :]).any(axis=1).astype(jnp.int32)

    mp_pad = jnp.pad(model_points, ((0, 0), (0, _NPAD - _N), (0, 0)))
    t_pad = jnp.pad(target, ((0, 0), (0, _NPAD - _N), (0, 0)),
                    constant_values=_PAD_COORD)
    mp3 = jnp.transpose(mp_pad, (2, 0, 1))
    tgt3 = jnp.transpose(t_pad, (2, 0, 1))

    out = _run(mask, rtv, mp3, tgt3)
    return out[:, 0]


# speedup vs baseline: 1.0006x; 1.0006x over previous
import jax
import jax.numpy as jnp
from jax.experimental import pallas as pl
from jax.experimental.pallas import tpu as pltpu

_B = 64
_N = 3000
_NPAD = 3072
_JT = 768
_NJT = _NPAD // _JT
_SYM = (12, 15, 18, 19, 20)
_PAD_COORD = 1.0e6


def _loss_kernel(mask_ref, rtv_ref, mp3_ref, tgt3_ref, out_ref,
                 tfx_s, tfy_s, tfz_s, r2_s):
    col_id = jax.lax.broadcasted_iota(jnp.int32, (1, _NPAD), 1)
    col_ok = col_id < _N
    lane = jnp.ones((1, 128), jnp.float32)

    mpx = mp3_ref[0]
    mpy = mp3_ref[1]
    mpz = mp3_ref[2]

    def c(k):
        return rtv_ref[:, k:k + 1]

    tfx = mpx * c(0) + mpy * c(3) + mpz * c(6) + c(9)
    tfy = mpx * c(1) + mpy * c(4) + mpz * c(7) + c(10)
    tfz = mpx * c(2) + mpy * c(5) + mpz * c(8) + c(11)
    tfx_s[...] = tfx
    tfy_s[...] = tfy
    tfz_s[...] = tfz

    tgx = tgt3_ref[0]
    tgy = tgt3_ref[1]
    tgz = tgt3_ref[2]
    r2_s[...] = tgx * tgx + tgy * tgy + tgz * tgz

    dx = tfx - tgx
    dy = tfy - tgy
    dz = tfz - tgz
    dis = jnp.sqrt(dx * dx + dy * dy + dz * dz)
    dsum = jnp.sum(jnp.where(col_ok, dis, 0.0), axis=1, keepdims=True) / _N
    out_ref[...] = dsum * lane

    def body(b, carry):
        @pl.when(mask_ref[b] != 0)
        def _sym_path():
            bs = pl.ds(b, 1)
            tfxr = tfx_s[bs, :]
            tfyr = tfy_s[bs, :]
            tfzr = tfz_s[bs, :]
            ones = jnp.ones_like(tfxr)
            q2 = tfxr * tfxr + tfyr * tfyr + tfzr * tfzr
            yk = jnp.concatenate(
                [-2.0 * tfxr, -2.0 * tfyr, -2.0 * tfzr, q2, ones], axis=0)
            tgxr = tgt3_ref[0, bs, :]
            tgyr = tgt3_ref[1, bs, :]
            tgzr = tgt3_ref[2, bs, :]
            r2r = r2_s[bs, :]
            half = _NPAD // 2
            ykl = yk[:, :half]
            ykr = yk[:, half:]
            dminl = None
            dminr = None
            for j in range(_NJT):
                js = slice(j * _JT, (j + 1) * _JT)
                xkT = jnp.concatenate(
                    [tgxr[:, js], tgyr[:, js], tgzr[:, js],
                     ones[:, js], r2r[:, js]], axis=0)
                d2l = jax.lax.dot_general(
                    xkT, ykl, (((0,), (0,)), ((), ())),
                    preferred_element_type=jnp.float32)
                d2r = jax.lax.dot_general(
                    xkT, ykr, (((0,), (0,)), ((), ())),
                    preferred_element_type=jnp.float32)
                tl = jnp.min(d2l, axis=0, keepdims=True)
                tr = jnp.min(d2r, axis=0, keepdims=True)
                dminl = tl if dminl is None else jnp.minimum(dminl, tl)
                dminr = tr if dminr is None else jnp.minimum(dminr, tr)
            dmin = jnp.concatenate([dminl, dminr], axis=1)
            disr = jnp.sqrt(jnp.maximum(dmin, 0.0))
            val = jnp.sum(jnp.where(col_ok, disr, 0.0)) / _N
            out_ref[bs, :] = val * lane

        return carry

    jax.lax.fori_loop(0, _B, body, 0)


@jax.jit
def _run(mask, rtv, mp3, tgt3):
    grid_spec = pltpu.PrefetchScalarGridSpec(
        num_scalar_prefetch=1,
        grid=(1,),
        in_specs=[
            pl.BlockSpec((_B, 16), lambda g, m: (0, 0)),
            pl.BlockSpec((3, _B, _NPAD), lambda g, m: (0, 0, 0)),
            pl.BlockSpec((3, _B, _NPAD), lambda g, m: (0, 0, 0)),
        ],
        out_specs=pl.BlockSpec((_B, 128), lambda g, m: (0, 0)),
        scratch_shapes=[pltpu.VMEM((_B, _NPAD), jnp.float32)] * 4,
    )
    return pl.pallas_call(
        _loss_kernel,
        grid_spec=grid_spec,
        out_shape=jax.ShapeDtypeStruct((_B, 128), jnp.float32),
        compiler_params=pltpu.CompilerParams(
            dimension_semantics=("arbitrary",),
        ),
    )(mask, rtv, mp3, tgt3)


def kernel(pred_r, pred_t, target, model_points, idx):
    pred_r = pred_r / jnp.linalg.norm(pred_r, axis=1, keepdims=True)
    w, x, y, z = pred_r[:, 0], pred_r[:, 1], pred_r[:, 2], pred_r[:, 3]
    rt = jnp.stack([
        1.0 - 2.0 * (y * y + z * z), 2.0 * (x * y - w * z), 2.0 * (x * z + w * y),
        2.0 * (x * y + w * z), 1.0 - 2.0 * (x * x + z * z), 2.0 * (y * z - w * x),
        2.0 * (x * z - w * y), 2.0 * (y * z + w * x), 1.0 - 2.0 * (x * x + y * y),
        pred_t[:, 0], pred_t[:, 1], pred_t[:, 2],
    ], axis=1)
    rtv = jnp.pad(rt, ((0, 0), (0, 4)))

    sym = jnp.asarray(_SYM, dtype=idx.dtype)
    mask = (idx[:, 0][:, None] == sym[None, :]).any(axis=1).astype(jnp.int32)

    mp_pad = jnp.pad(model_points, ((0, 0), (0, _NPAD - _N), (0, 0)))
    t_pad = jnp.pad(target, ((0, 0), (0, _NPAD - _N), (0, 0)),
                    constant_values=_PAD_COORD)
    mp3 = jnp.transpose(mp_pad, (2, 0, 1))
    tgt3 = jnp.transpose(t_pad, (2, 0, 1))

    out = _run(mask, rtv, mp3, tgt3)
    return out[:, 0]
